# Initial kernel scaffold; baseline (speedup 1.0000x reference)
import jax
import jax.numpy as jnp
from jax.experimental import pallas as pl


def kernel(x_0, inc_node, inc_hedge, query_node_indices, cell_member_node, cell_member_cell, W1, b1, W2, b2, phi_W, phi_b, rho_W, rho_b, qp_W, qp_b, g1_W, g1_b, g2_W, g2_b, mlp1_W, mlp1_b, mlp2_W, mlp2_b):
    raise NotImplementedError("write your pallas kernel here")


# speedup vs baseline: 1.9261x; 1.9261x over previous
import functools

import jax
import jax.numpy as jnp
from jax import lax
from jax.experimental import pallas as pl
from jax.experimental.pallas import tpu as pltpu
from jax.experimental.pallas import tpu_sc as plsc

_N_NODES = 10000
_N_HEDGES = 20000
_N_INC = 320000
_D = 128
_HID = 256
_Q = 64
_M = 512
_N_MEM = 4096

_F = 64
_K = 80
_ZR = 125
_N_TILES = 16
_PER_TILE = _N_INC // _N_TILES
_CHUNKS = _PER_TILE // _K
_EROWS = _N_HEDGES // _N_TILES
_NROWS = _N_NODES // _N_TILES
_WROWS = 624
_WTAIL = _N_NODES - 16 * _WROWS
_EWROWS = 1248
_EWTAIL = _N_HEDGES - 16 * _EWROWS

_mesh = plsc.VectorSubcoreMesh(core_axis_name="c", subcore_axis_name="s",
                               num_cores=2, num_subcores=16)


def _make_hconv(n_slices):
    n_passes = n_slices // 2

    @functools.partial(
        pl.kernel,
        out_type=jax.ShapeDtypeStruct((n_slices * _N_HEDGES, _F), jnp.float32),
        mesh=_mesh,
        scratch_types=[
            pltpu.VMEM((_K,), jnp.int32),
            pltpu.VMEM((_K,), jnp.int32),
            pltpu.VMEM((_K, _F), jnp.float32),
            pltpu.VMEM((_ZR, _F), jnp.float32),
            pltpu.VMEM_SHARED((_N_HEDGES, _F), jnp.float32),
            pltpu.SemaphoreType.DMA,
        ],
        compiler_params=pltpu.CompilerParams(use_tc_tiling_on_sc=False),
    )
    def hconv_s1(xs_ref, incn_ref, inch_ref, agge_hbm,
                 idxn_v, idxh_v, rows_v, zero_v, agge_sh, sem):
        c = lax.axis_index("c")
        s = lax.axis_index("s")

        def zinit(i, carry):
            for f4 in range(_F // 16):
                zero_v[i, pl.ds(f4 * 16, 16)] = jnp.zeros((16,), jnp.float32)
            return carry
        lax.fori_loop(0, _ZR, zinit, 0)

        for p in range(n_passes):
            slice_id = 2 * p + c
            for r in range(_EROWS // _ZR):
                pltpu.sync_copy(
                    zero_v, agge_sh.at[pl.ds(s * _EROWS + r * _ZR, _ZR)])
            plsc.subcore_barrier()

            noff = slice_id * _N_NODES

            def s1(k, carry):
                base = pl.multiple_of(s * _PER_TILE + k * _K, 8)
                pltpu.sync_copy(incn_ref.at[pl.ds(base, _K)], idxn_v)
                pltpu.sync_copy(inch_ref.at[pl.ds(base, _K)], idxh_v)
                for j in range(_K // 16):
                    idxn_v[pl.ds(j * 16, 16)] = (
                        idxn_v[pl.ds(j * 16, 16)] + noff)
                pltpu.async_copy(xs_ref.at[idxn_v], rows_v, sem).wait()
                pltpu.sync_copy(rows_v, agge_sh.at[idxh_v], add=True)
                return carry
            lax.fori_loop(0, _CHUNKS, s1, 0)
            plsc.subcore_barrier()

            pltpu.sync_copy(
                agge_sh.at[pl.ds(s * _EWROWS, _EWROWS)],
                agge_hbm.at[pl.ds(slice_id * _N_HEDGES + s * _EWROWS,
                                  _EWROWS)])

            @pl.when(s == 0)
            def _():
                pltpu.sync_copy(
                    agge_sh.at[pl.ds(16 * _EWROWS, _EWTAIL)],
                    agge_hbm.at[pl.ds(slice_id * _N_HEDGES + 16 * _EWROWS,
                                      _EWTAIL)])
            plsc.subcore_barrier()

    @functools.partial(
        pl.kernel,
        out_type=jax.ShapeDtypeStruct((n_slices * _N_NODES, _F), jnp.float32),
        mesh=_mesh,
        scratch_types=[
            pltpu.VMEM((_K,), jnp.int32),
            pltpu.VMEM((_K,), jnp.int32),
            pltpu.VMEM((_K, _F), jnp.float32),
            pltpu.VMEM((_ZR, _F), jnp.float32),
            pltpu.VMEM_SHARED((_N_NODES, _F), jnp.float32),
            pltpu.SemaphoreType.DMA,
        ],
        compiler_params=pltpu.CompilerParams(use_tc_tiling_on_sc=False),
    )
    def hconv_s2(agge_hbm, incn_ref, inch_ref, out_ref,
                 idxn_v, idxh_v, rows_v, zero_v, aggn_sh, sem):
        c = lax.axis_index("c")
        s = lax.axis_index("s")

        def zinit(i, carry):
            for f4 in range(_F // 16):
                zero_v[i, pl.ds(f4 * 16, 16)] = jnp.zeros((16,), jnp.float32)
            return carry
        lax.fori_loop(0, _ZR, zinit, 0)

        for p in range(n_passes):
            slice_id = 2 * p + c
            for r in range(_NROWS // _ZR):
                pltpu.sync_copy(
                    zero_v, aggn_sh.at[pl.ds(s * _NROWS + r * _ZR, _ZR)])
            plsc.subcore_barrier()

            hoff = slice_id * _N_HEDGES

            def s2(k, carry):
                base = pl.multiple_of(s * _PER_TILE + k * _K, 8)
                pltpu.sync_copy(inch_ref.at[pl.ds(base, _K)], idxh_v)
                pltpu.sync_copy(incn_ref.at[pl.ds(base, _K)], idxn_v)
                for j in range(_K // 16):
                    idxh_v[pl.ds(j * 16, 16)] = (
                        idxh_v[pl.ds(j * 16, 16)] + hoff)
                pltpu.async_copy(agge_hbm.at[idxh_v], rows_v, sem).wait()
                pltpu.sync_copy(rows_v, aggn_sh.at[idxn_v], add=True)
                return carry
            lax.fori_loop(0, _CHUNKS, s2, 0)
            plsc.subcore_barrier()

            pltpu.sync_copy(
                aggn_sh.at[pl.ds(s * _WROWS, _WROWS)],
                out_ref.at[pl.ds(slice_id * _N_NODES + s * _WROWS, _WROWS)])

            @pl.when(s == 0)
            def _():
                pltpu.sync_copy(
                    aggn_sh.at[pl.ds(16 * _WROWS, _WTAIL)],
                    out_ref.at[pl.ds(slice_id * _N_NODES + 16 * _WROWS,
                                     _WTAIL)])
            plsc.subcore_barrier()

    def hconv(xs, incn, inch):
        agge = hconv_s1(xs, incn, inch)
        return hconv_s2(agge, incn, inch)

    hconv.s1 = hconv_s1
    hconv.s2 = hconv_s2
    return hconv


_hconv2 = _make_hconv(2)
_hconv4 = _make_hconv(4)


_CG = _N_MEM // 32
_QG = 16


@functools.partial(
    pl.kernel,
    out_type=[
        jax.ShapeDtypeStruct((_N_MEM, _D), jnp.float32),
        jax.ShapeDtypeStruct((_Q, _D), jnp.float32),
    ],
    mesh=_mesh,
    scratch_types=[
        pltpu.VMEM((_CG,), jnp.int32),
        pltpu.VMEM((_CG, _D), jnp.float32),
        pltpu.VMEM((_QG,), jnp.int32),
        pltpu.VMEM((_QG, _D), jnp.float32),
        pltpu.SemaphoreType.DMA,
    ],
)
def _gather_rows(table_ref, cidx_ref, qidx_ref, outc_ref, outq_ref,
                 cidx_v, crow_v, qidx_v, qrow_v, sem):
    c = lax.axis_index("c")
    s = lax.axis_index("s")
    wid = s * 2 + c
    pltpu.sync_copy(cidx_ref.at[pl.ds(wid * _CG, _CG)], cidx_v)
    pltpu.async_copy(table_ref.at[cidx_v], crow_v, sem).wait()
    pltpu.sync_copy(crow_v, outc_ref.at[pl.ds(wid * _CG, _CG)])

    @pl.when(wid < _Q // _QG)
    def _():
        pltpu.sync_copy(qidx_ref.at[pl.ds(wid * _QG, _QG)], qidx_v)
        pltpu.async_copy(table_ref.at[qidx_v], qrow_v, sem).wait()
        pltpu.sync_copy(qrow_v, outq_ref.at[pl.ds(wid * _QG, _QG)])


_R = 1000


def _tc1_body(x_ref, w_ref, b_ref, out_ref):
    xv = x_ref[...]
    w = w_ref[...]
    acc = (jnp.dot(xv[0], w[0:_F, :], preferred_element_type=jnp.float32)
           + jnp.dot(xv[1], w[_F:2 * _F, :], preferred_element_type=jnp.float32)
           + b_ref[...])
    h = jnp.maximum(acc, 0.0)
    for sl in range(4):
        out_ref[sl] = h[:, sl * _F:(sl + 1) * _F]


def _tc1(aggn1, W1, b1):
    x3 = aggn1.reshape(2, _N_NODES, _F)
    return pl.pallas_call(
        _tc1_body,
        grid=(_N_NODES // _R,),
        in_specs=[
            pl.BlockSpec((2, _R, _F), lambda i: (0, i, 0)),
            pl.BlockSpec((_D, _HID), lambda i: (0, 0)),
            pl.BlockSpec((1, _HID), lambda i: (0, 0)),
        ],
        out_specs=pl.BlockSpec((4, _R, _F), lambda i: (0, i, 0)),
        out_shape=jax.ShapeDtypeStruct((4, _N_NODES, _F), jnp.float32),
    )(x3, W1, b1.reshape(1, _HID))


def _tc2_body(x_ref, w_ref, b_ref, out_ref):
    xv = x_ref[...]
    w = w_ref[...]
    acc = b_ref[...]
    for sl in range(4):
        acc = acc + jnp.dot(xv[sl], w[sl * _F:(sl + 1) * _F, :],
                            preferred_element_type=jnp.float32)
    out_ref[...] = jnp.maximum(acc, 0.0)


def _tc2(aggn2, W2, b2):
    x3 = aggn2.reshape(4, _N_NODES, _F)
    return pl.pallas_call(
        _tc2_body,
        grid=(_N_NODES // _R,),
        in_specs=[
            pl.BlockSpec((4, _R, _F), lambda i: (0, i, 0)),
            pl.BlockSpec((_HID, _D), lambda i: (0, 0)),
            pl.BlockSpec((1, _D), lambda i: (0, 0)),
        ],
        out_specs=pl.BlockSpec((_R, _D), lambda i: (i, 0)),
        out_shape=jax.ShapeDtypeStruct((_N_NODES, _D), jnp.float32),
    )(x3, W2, b2.reshape(1, _D))


def _score_body(x0q_ref, rfq_ref, x0m_ref, rfm_ref, ids_ref,
                phiW_ref, phib_ref, rhoW_ref, rhob_ref,
                qpW_ref, qpb_ref, g1W_ref, g1b_ref, g2W_ref, g2b_ref,
                m1W_ref, m1b_ref, m2W_ref, m2b_ref, out_ref):
    qpW = qpW_ref[...]
    qpb = qpb_ref[...]
    base_q = jnp.dot(x0q_ref[...], qpW, preferred_element_type=jnp.float32) + qpb
    topo_q = jnp.dot(rfq_ref[...], qpW, preferred_element_type=jnp.float32) + qpb

    gh = jnp.maximum(
        jnp.dot(topo_q, g1W_ref[...], preferred_element_type=jnp.float32)
        + g1b_ref[...], 0.0)
    glog = jnp.dot(gh, g2W_ref[...], preferred_element_type=jnp.float32) + g2b_ref[...]
    gate = 1.0 / (1.0 + jnp.exp(-glog))

    phiW = phiW_ref[...]
    phib = phib_ref[...]
    hb = jnp.maximum(
        jnp.dot(x0m_ref[...], phiW, preferred_element_type=jnp.float32) + phib, 0.0)
    hr = jnp.maximum(
        jnp.dot(rfm_ref[...], phiW, preferred_element_type=jnp.float32) + phib, 0.0)

    mi = lax.broadcasted_iota(jnp.int32, (_M, _N_MEM), 0)
    sel = (mi == ids_ref[...]).astype(jnp.float32)
    pooled_b = jnp.dot(sel, hb, preferred_element_type=jnp.float32)
    pooled_r = jnp.dot(sel, hr, preferred_element_type=jnp.float32)

    rhoW = rhoW_ref[...]
    rhob = rhob_ref[...]
    base_c = jnp.dot(pooled_b, rhoW, preferred_element_type=jnp.float32) + rhob
    topo_c = jnp.dot(pooled_r, rhoW, preferred_element_type=jnp.float32) + rhob

    m1W = m1W_ref[...]
    q_final = (1.0 - gate) * base_q + gate * topo_q
    qh = jnp.dot(q_final, m1W[:_HID, :], preferred_element_type=jnp.float32) + m1b_ref[...]
    BCc = jnp.dot(base_c, m1W[_HID:, :], preferred_element_type=jnp.float32)
    TCc = jnp.dot(topo_c, m1W[_HID:, :], preferred_element_type=jnp.float32)

    w2 = m2W_ref[...][:, 0]
    b2s = m2b_ref[0, 0]
    QB = 8
    for qb in range(_Q // QB):
        lo, hi = qb * QB, (qb + 1) * QB
        g = gate[lo:hi]
        hq = (qh[lo:hi][:, None, :]
              + (1.0 - g)[:, :, None] * BCc[None, :, :]
              + g[:, :, None] * TCc[None, :, :])
        sc = jnp.sum(jnp.maximum(hq, 0.0) * w2[None, None, :], axis=-1) + b2s
        out_ref[lo:hi] = sc


def _score(x0q, rfq, x0m, rfm, ids, phi_W, phi_b, rho_W, rho_b,
           qp_W, qp_b, g1_W, g1_b, g2_W, g2_b, mlp1_W, mlp1_b, mlp2_W, mlp2_b):
    return pl.pallas_call(
        _score_body,
        out_shape=jax.ShapeDtypeStruct((_Q, _M), jnp.float32),
    )(x0q, rfq, x0m, rfm, ids.reshape(1, _N_MEM),
      phi_W, phi_b.reshape(1, _HID), rho_W, rho_b.reshape(1, _HID),
      qp_W, qp_b.reshape(1, _HID), g1_W, g1_b.reshape(1, _HID),
      g2_W, g2_b.reshape(1, 1),
      mlp1_W, mlp1_b.reshape(1, _HID), mlp2_W, mlp2_b.reshape(1, 1))


def kernel(x_0, inc_node, inc_hedge, query_node_indices, cell_member_node,
           cell_member_cell, W1, b1, W2, b2, phi_W, phi_b, rho_W, rho_b,
           qp_W, qp_b, g1_W, g1_b, g2_W, g2_b, mlp1_W, mlp1_b, mlp2_W, mlp2_b):
    incn = inc_node.astype(jnp.int32)
    inch = inc_hedge.astype(jnp.int32)
    cidx = cell_member_node.astype(jnp.int32)
    qidx = query_node_indices.astype(jnp.int32)
    ids = cell_member_cell.astype(jnp.int32)

    xs1 = x_0.reshape(_N_NODES, 2, _F).transpose(1, 0, 2).reshape(2 * _N_NODES, _F)
    aggn1 = _hconv2(xs1, incn, inch)
    h1s = _tc1(aggn1, W1, b1).reshape(4 * _N_NODES, _F)
    aggn2 = _hconv4(h1s, incn, inch)
    refined = _tc2(aggn2, W2, b2)

    x0m, x0q = _gather_rows(x_0, cidx, qidx)
    rfm, rfq = _gather_rows(refined, cidx, qidx)

    return _score(x0q, rfq, x0m, rfm, ids, phi_W, phi_b, rho_W, rho_b,
                  qp_W, qp_b, g1_W, g1_b, g2_W, g2_b,
                  mlp1_W, mlp1_b, mlp2_W, mlp2_b)


# speedup vs baseline: 6.3879x; 3.3164x over previous
import functools

import jax
import jax.numpy as jnp
from jax import lax
from jax.experimental import pallas as pl
from jax.experimental.pallas import tpu as pltpu
from jax.experimental.pallas import tpu_sc as plsc

_N_NODES = 10000
_N_HEDGES = 20000
_N_INC = 320000
_D = 128
_HID = 256
_Q = 64
_M = 512
_N_MEM = 4096

_F = 64
_K = 80
_ZR = 25
_N_TILES = 16
_PER_TILE = _N_INC // _N_TILES
_CHUNKS = _PER_TILE // _K
_EROWS = _N_HEDGES // _N_TILES
_NROWS = _N_NODES // _N_TILES
_WROWS = 624
_WTAIL = _N_NODES - 16 * _WROWS
_EWROWS = 1248
_EWTAIL = _N_HEDGES - 16 * _EWROWS

_mesh = plsc.VectorSubcoreMesh(core_axis_name="c", subcore_axis_name="s",
                               num_cores=2, num_subcores=16)


_DR = 5
_NBLK = 5
_BCHUNKS = _CHUNKS // _NBLK
_BIDX = _BCHUNKS * _K
_BITERS = _BCHUNKS // _DR - 1


def _zero_acc(zero_v, acc_sh, s, rows_per_tile, msem):
    nch = rows_per_tile // _ZR
    for r in range(nch):
        pltpu.async_copy(
            zero_v, acc_sh.at[pl.ds(s * rows_per_tile + r * _ZR, _ZR)], msem)
    for r in range(nch):
        pltpu.make_async_copy(
            zero_v, acc_sh.at[pl.ds(s * rows_per_tile + r * _ZR, _ZR)],
            msem).wait()


def _adjust_idx(idx_1d, off):
    def body(i, carry):
        idx_1d[pl.ds(i * 16, 16)] = idx_1d[pl.ds(i * 16, 16)] + off
        return carry
    lax.fori_loop(0, _BIDX // 16, body, 0)


def _spread_idx(idx_1d, idx_2d):
    def body(k, carry):
        for j in range(_K // 16):
            idx_2d[k, pl.ds(j * 16, 16)] = idx_1d[pl.ds(k * _K + j * 16, 16)]
        return carry
    lax.fori_loop(0, _BCHUNKS, body, 0)


def _stage(table_ref, acc_sh, gidx_hbm, sidx_hbm, goff, s,
           g1d, s1d, s2d, rows_v, gsems, ssems):
    def gather(k, b):
        return pltpu.async_copy(
            table_ref.at[g1d.at[pl.ds(k * _K, _K)]], rows_v.at[b], gsems[b])

    def gwait(k, b):
        pltpu.make_async_copy(
            table_ref.at[g1d.at[pl.ds(k * _K, _K)]], rows_v.at[b],
            gsems[b]).wait()

    def scat(k, b):
        return pltpu.async_copy(
            rows_v.at[b], acc_sh.at[s2d.at[k]], ssems[b], add=True)

    def swait(k, b):
        pltpu.make_async_copy(
            rows_v.at[b], acc_sh.at[s2d.at[k]], ssems[b]).wait()

    for blk in range(_NBLK):
        base = pl.multiple_of(s * _PER_TILE + blk * _BIDX, 8)
        pltpu.sync_copy(gidx_hbm.at[pl.ds(base, _BIDX)], g1d)
        pltpu.sync_copy(sidx_hbm.at[pl.ds(base, _BIDX)], s1d)
        _adjust_idx(g1d, goff)
        _spread_idx(s1d, s2d)

        for b in range(_DR):
            gather(b, b)

        def body(i, carry):
            k0 = i * _DR
            for b in range(_DR):
                gwait(k0 + b, b)
                scat(k0 + b, b)
            for b in range(_DR):
                swait(k0 + b, b)
                gather(k0 + _DR + b, b)
            return carry
        lax.fori_loop(0, _BITERS, body, 0)

        kl = _BITERS * _DR
        for b in range(_DR):
            gwait(kl + b, b)
            scat(kl + b, b)
        for b in range(_DR):
            swait(kl + b, b)


def _make_hconv(n_slices):
    n_passes = n_slices // 2

    sem_types = [pltpu.SemaphoreType.DMA] * (2 * _DR + 1)

    @functools.partial(
        pl.kernel,
        out_type=jax.ShapeDtypeStruct((n_slices * _N_HEDGES, _F), jnp.float32),
        mesh=_mesh,
        scratch_types=[
            pltpu.VMEM((_BIDX,), jnp.int32),
            pltpu.VMEM((_BIDX,), jnp.int32),
            pltpu.VMEM((_BCHUNKS, _K), jnp.int32),
            pltpu.VMEM((_DR, _K, _F), jnp.float32),
            pltpu.VMEM((_ZR, _F), jnp.float32),
            pltpu.VMEM_SHARED((_N_HEDGES, _F), jnp.float32),
        ] + sem_types,
        compiler_params=pltpu.CompilerParams(use_tc_tiling_on_sc=False),
    )
    def hconv_s1(xs_ref, incn_ref, inch_ref, agge_hbm,
                 gidx_1d, sidx_1d, sidx_2d, rows_v, zero_v, agge_sh, *sems):
        c = lax.axis_index("c")
        s = lax.axis_index("s")
        gsems, ssems, msem = sems[:_DR], sems[_DR:2 * _DR], sems[2 * _DR]

        def zinit(i, carry):
            for f4 in range(_F // 16):
                zero_v[i, pl.ds(f4 * 16, 16)] = jnp.zeros((16,), jnp.float32)
            return carry
        lax.fori_loop(0, _ZR, zinit, 0)

        for p in range(n_passes):
            slice_id = 2 * p + c
            _zero_acc(zero_v, agge_sh, s, _EROWS, msem)
            plsc.subcore_barrier()

            _stage(xs_ref, agge_sh, incn_ref, inch_ref,
                   slice_id * _N_NODES, s,
                   gidx_1d, sidx_1d, sidx_2d, rows_v, gsems, ssems)
            plsc.subcore_barrier()

            pltpu.sync_copy(
                agge_sh.at[pl.ds(s * _EWROWS, _EWROWS)],
                agge_hbm.at[pl.ds(slice_id * _N_HEDGES + s * _EWROWS,
                                  _EWROWS)])

            @pl.when(s == 0)
            def _():
                pltpu.sync_copy(
                    agge_sh.at[pl.ds(16 * _EWROWS, _EWTAIL)],
                    agge_hbm.at[pl.ds(slice_id * _N_HEDGES + 16 * _EWROWS,
                                      _EWTAIL)])
            plsc.subcore_barrier()

    @functools.partial(
        pl.kernel,
        out_type=jax.ShapeDtypeStruct((n_slices * _N_NODES, _F), jnp.float32),
        mesh=_mesh,
        scratch_types=[
            pltpu.VMEM((_BIDX,), jnp.int32),
            pltpu.VMEM((_BIDX,), jnp.int32),
            pltpu.VMEM((_BCHUNKS, _K), jnp.int32),
            pltpu.VMEM((_DR, _K, _F), jnp.float32),
            pltpu.VMEM((_ZR, _F), jnp.float32),
            pltpu.VMEM_SHARED((_N_NODES, _F), jnp.float32),
        ] + sem_types,
        compiler_params=pltpu.CompilerParams(use_tc_tiling_on_sc=False),
    )
    def hconv_s2(agge_hbm, incn_ref, inch_ref, out_ref,
                 gidx_1d, sidx_1d, sidx_2d, rows_v, zero_v, aggn_sh, *sems):
        c = lax.axis_index("c")
        s = lax.axis_index("s")
        gsems, ssems, msem = sems[:_DR], sems[_DR:2 * _DR], sems[2 * _DR]

        def zinit(i, carry):
            for f4 in range(_F // 16):
                zero_v[i, pl.ds(f4 * 16, 16)] = jnp.zeros((16,), jnp.float32)
            return carry
        lax.fori_loop(0, _ZR, zinit, 0)

        for p in range(n_passes):
            slice_id = 2 * p + c
            _zero_acc(zero_v, aggn_sh, s, _NROWS, msem)
            plsc.subcore_barrier()

            _stage(agge_hbm, aggn_sh, inch_ref, incn_ref,
                   slice_id * _N_HEDGES, s,
                   gidx_1d, sidx_1d, sidx_2d, rows_v, gsems, ssems)
            plsc.subcore_barrier()

            pltpu.sync_copy(
                aggn_sh.at[pl.ds(s * _WROWS, _WROWS)],
                out_ref.at[pl.ds(slice_id * _N_NODES + s * _WROWS, _WROWS)])

            @pl.when(s == 0)
            def _():
                pltpu.sync_copy(
                    aggn_sh.at[pl.ds(16 * _WROWS, _WTAIL)],
                    out_ref.at[pl.ds(slice_id * _N_NODES + 16 * _WROWS,
                                     _WTAIL)])
            plsc.subcore_barrier()

    def hconv(xs, incn3, inch3):
        agge = hconv_s1(xs, incn3, inch3)
        return hconv_s2(agge, incn3, inch3)

    hconv.s1 = hconv_s1
    hconv.s2 = hconv_s2
    return hconv


_hconv2 = _make_hconv(2)
_hconv4 = _make_hconv(4)


_CG = _N_MEM // 32
_QG = 16


@functools.partial(
    pl.kernel,
    out_type=[
        jax.ShapeDtypeStruct((_N_MEM, _D), jnp.float32),
        jax.ShapeDtypeStruct((_Q, _D), jnp.float32),
    ],
    mesh=_mesh,
    scratch_types=[
        pltpu.VMEM((_CG,), jnp.int32),
        pltpu.VMEM((_CG, _D), jnp.float32),
        pltpu.VMEM((_QG,), jnp.int32),
        pltpu.VMEM((_QG, _D), jnp.float32),
        pltpu.SemaphoreType.DMA,
    ],
)
def _gather_rows(table_ref, cidx_ref, qidx_ref, outc_ref, outq_ref,
                 cidx_v, crow_v, qidx_v, qrow_v, sem):
    c = lax.axis_index("c")
    s = lax.axis_index("s")
    wid = s * 2 + c
    pltpu.sync_copy(cidx_ref.at[pl.ds(wid * _CG, _CG)], cidx_v)
    pltpu.async_copy(table_ref.at[cidx_v], crow_v, sem).wait()
    pltpu.sync_copy(crow_v, outc_ref.at[pl.ds(wid * _CG, _CG)])

    @pl.when(wid < _Q // _QG)
    def _():
        pltpu.sync_copy(qidx_ref.at[pl.ds(wid * _QG, _QG)], qidx_v)
        pltpu.async_copy(table_ref.at[qidx_v], qrow_v, sem).wait()
        pltpu.sync_copy(qrow_v, outq_ref.at[pl.ds(wid * _QG, _QG)])


_R = 1000


def _tc1_body(x_ref, w_ref, b_ref, out_ref):
    xv = x_ref[...]
    w = w_ref[...]
    acc = (jnp.dot(xv[0], w[0:_F, :], preferred_element_type=jnp.float32)
           + jnp.dot(xv[1], w[_F:2 * _F, :], preferred_element_type=jnp.float32)
           + b_ref[...])
    h = jnp.maximum(acc, 0.0)
    for sl in range(4):
        out_ref[sl] = h[:, sl * _F:(sl + 1) * _F]


def _tc1(aggn1, W1, b1):
    x3 = aggn1.reshape(2, _N_NODES, _F)
    return pl.pallas_call(
        _tc1_body,
        grid=(_N_NODES // _R,),
        in_specs=[
            pl.BlockSpec((2, _R, _F), lambda i: (0, i, 0)),
            pl.BlockSpec((_D, _HID), lambda i: (0, 0)),
            pl.BlockSpec((1, _HID), lambda i: (0, 0)),
        ],
        out_specs=pl.BlockSpec((4, _R, _F), lambda i: (0, i, 0)),
        out_shape=jax.ShapeDtypeStruct((4, _N_NODES, _F), jnp.float32),
    )(x3, W1, b1.reshape(1, _HID))


def _tc2_body(x_ref, w_ref, b_ref, out_ref):
    xv = x_ref[...]
    w = w_ref[...]
    acc = b_ref[...]
    for sl in range(4):
        acc = acc + jnp.dot(xv[sl], w[sl * _F:(sl + 1) * _F, :],
                            preferred_element_type=jnp.float32)
    out_ref[...] = jnp.maximum(acc, 0.0)


def _tc2(aggn2, W2, b2):
    x3 = aggn2.reshape(4, _N_NODES, _F)
    return pl.pallas_call(
        _tc2_body,
        grid=(_N_NODES // _R,),
        in_specs=[
            pl.BlockSpec((4, _R, _F), lambda i: (0, i, 0)),
            pl.BlockSpec((_HID, _D), lambda i: (0, 0)),
            pl.BlockSpec((1, _D), lambda i: (0, 0)),
        ],
        out_specs=pl.BlockSpec((_R, _D), lambda i: (i, 0)),
        out_shape=jax.ShapeDtypeStruct((_N_NODES, _D), jnp.float32),
    )(x3, W2, b2.reshape(1, _D))


def _score_body(x0q_ref, rfq_ref, x0m_ref, rfm_ref, ids_ref,
                phiW_ref, phib_ref, rhoW_ref, rhob_ref,
                qpW_ref, qpb_ref, g1W_ref, g1b_ref, g2W_ref, g2b_ref,
                m1W_ref, m1b_ref, m2W_ref, m2b_ref, out_ref):
    qpW = qpW_ref[...]
    qpb = qpb_ref[...]
    base_q = jnp.dot(x0q_ref[...], qpW, preferred_element_type=jnp.float32) + qpb
    topo_q = jnp.dot(rfq_ref[...], qpW, preferred_element_type=jnp.float32) + qpb

    gh = jnp.maximum(
        jnp.dot(topo_q, g1W_ref[...], preferred_element_type=jnp.float32)
        + g1b_ref[...], 0.0)
    glog = jnp.dot(gh, g2W_ref[...], preferred_element_type=jnp.float32) + g2b_ref[...]
    gate = 1.0 / (1.0 + jnp.exp(-glog))

    phiW = phiW_ref[...]
    phib = phib_ref[...]
    hb = jnp.maximum(
        jnp.dot(x0m_ref[...], phiW, preferred_element_type=jnp.float32) + phib, 0.0)
    hr = jnp.maximum(
        jnp.dot(rfm_ref[...], phiW, preferred_element_type=jnp.float32) + phib, 0.0)

    mi = lax.broadcasted_iota(jnp.int32, (_M, _N_MEM), 0)
    sel = (mi == ids_ref[...]).astype(jnp.float32)
    pooled_b = jnp.dot(sel, hb, preferred_element_type=jnp.float32)
    pooled_r = jnp.dot(sel, hr, preferred_element_type=jnp.float32)

    rhoW = rhoW_ref[...]
    rhob = rhob_ref[...]
    base_c = jnp.dot(pooled_b, rhoW, preferred_element_type=jnp.float32) + rhob
    topo_c = jnp.dot(pooled_r, rhoW, preferred_element_type=jnp.float32) + rhob

    m1W = m1W_ref[...]
    q_final = (1.0 - gate) * base_q + gate * topo_q
    qh = jnp.dot(q_final, m1W[:_HID, :], preferred_element_type=jnp.float32) + m1b_ref[...]
    BCc = jnp.dot(base_c, m1W[_HID:, :], preferred_element_type=jnp.float32)
    TCc = jnp.dot(topo_c, m1W[_HID:, :], preferred_element_type=jnp.float32)

    w2 = m2W_ref[...][:, 0]
    b2s = m2b_ref[0, 0]
    QB = 8
    for qb in range(_Q // QB):
        lo, hi = qb * QB, (qb + 1) * QB
        g = gate[lo:hi]
        hq = (qh[lo:hi][:, None, :]
              + (1.0 - g)[:, :, None] * BCc[None, :, :]
              + g[:, :, None] * TCc[None, :, :])
        sc = jnp.sum(jnp.maximum(hq, 0.0) * w2[None, None, :], axis=-1) + b2s
        out_ref[lo:hi] = sc


def _score(x0q, rfq, x0m, rfm, ids, phi_W, phi_b, rho_W, rho_b,
           qp_W, qp_b, g1_W, g1_b, g2_W, g2_b, mlp1_W, mlp1_b, mlp2_W, mlp2_b):
    return pl.pallas_call(
        _score_body,
        out_shape=jax.ShapeDtypeStruct((_Q, _M), jnp.float32),
    )(x0q, rfq, x0m, rfm, ids.reshape(1, _N_MEM),
      phi_W, phi_b.reshape(1, _HID), rho_W, rho_b.reshape(1, _HID),
      qp_W, qp_b.reshape(1, _HID), g1_W, g1_b.reshape(1, _HID),
      g2_W, g2_b.reshape(1, 1),
      mlp1_W, mlp1_b.reshape(1, _HID), mlp2_W, mlp2_b.reshape(1, 1))


def kernel(x_0, inc_node, inc_hedge, query_node_indices, cell_member_node,
           cell_member_cell, W1, b1, W2, b2, phi_W, phi_b, rho_W, rho_b,
           qp_W, qp_b, g1_W, g1_b, g2_W, g2_b, mlp1_W, mlp1_b, mlp2_W, mlp2_b):
    incn = inc_node.astype(jnp.int32)
    inch = inc_hedge.astype(jnp.int32)
    cidx = cell_member_node.astype(jnp.int32)
    qidx = query_node_indices.astype(jnp.int32)
    ids = cell_member_cell.astype(jnp.int32)

    xs1 = x_0.reshape(_N_NODES, 2, _F).transpose(1, 0, 2).reshape(2 * _N_NODES, _F)
    aggn1 = _hconv2(xs1, incn, inch)
    h1s = _tc1(aggn1, W1, b1).reshape(4 * _N_NODES, _F)
    aggn2 = _hconv4(h1s, incn, inch)
    refined = _tc2(aggn2, W2, b2)

    x0m, x0q = _gather_rows(x_0, cidx, qidx)
    rfm, rfq = _gather_rows(refined, cidx, qidx)

    return _score(x0q, rfq, x0m, rfm, ids, phi_W, phi_b, rho_W, rho_b,
                  qp_W, qp_b, g1_W, g1_b, g2_W, g2_b,
                  mlp1_W, mlp1_b, mlp2_W, mlp2_b)


# speedup vs baseline: 8.0785x; 1.2646x over previous
import functools

import jax
import jax.numpy as jnp
from jax import lax
from jax.experimental import pallas as pl
from jax.experimental.pallas import tpu as pltpu
from jax.experimental.pallas import tpu_sc as plsc

_N_NODES = 10000
_N_HEDGES = 20000
_N_INC = 320000
_D = 128
_HID = 256
_Q = 64
_M = 512
_N_MEM = 4096

_F = 64
_K = 80
_ZR = 25
_N_TILES = 16
_PER_TILE = _N_INC // _N_TILES
_CHUNKS = _PER_TILE // _K
_EROWS = _N_HEDGES // _N_TILES
_NROWS = _N_NODES // _N_TILES
_WROWS = 624
_WTAIL = _N_NODES - 16 * _WROWS
_EWROWS = 1248
_EWTAIL = _N_HEDGES - 16 * _EWROWS

_mesh = plsc.VectorSubcoreMesh(core_axis_name="c", subcore_axis_name="s",
                               num_cores=2, num_subcores=16)


_DR = 5
_NBLK = 5
_BCHUNKS = _CHUNKS // _NBLK
_BIDX = _BCHUNKS * _K
_BITERS = _BCHUNKS // _DR - 1


def _zero_acc(zero_v, acc_sh, s, rows_per_tile, msem):
    nch = rows_per_tile // _ZR
    for r in range(nch):
        pltpu.async_copy(
            zero_v, acc_sh.at[pl.ds(s * rows_per_tile + r * _ZR, _ZR)], msem)
    for r in range(nch):
        pltpu.make_async_copy(
            zero_v, acc_sh.at[pl.ds(s * rows_per_tile + r * _ZR, _ZR)],
            msem).wait()


def _adjust_idx(idx_1d, off):
    def body(i, carry):
        idx_1d[pl.ds(i * 16, 16)] = idx_1d[pl.ds(i * 16, 16)] + off
        return carry
    lax.fori_loop(0, _BIDX // 16, body, 0)


def _spread_idx(idx_1d, idx_2d):
    def body(k, carry):
        for j in range(_K // 16):
            idx_2d[k, pl.ds(j * 16, 16)] = idx_1d[pl.ds(k * _K + j * 16, 16)]
        return carry
    lax.fori_loop(0, _BCHUNKS, body, 0)


def _stage(table_ref, acc_sh, gidx_hbm, sidx_hbm, goff, s,
           g1d, s1d, s2d, rows_v, gsems, ssems):
    def gather(k, b):
        return pltpu.async_copy(
            table_ref.at[g1d.at[pl.ds(k * _K, _K)]], rows_v.at[b], gsems[b])

    def gwait(k, b):
        pltpu.make_async_copy(
            table_ref.at[g1d.at[pl.ds(k * _K, _K)]], rows_v.at[b],
            gsems[b]).wait()

    def scat(k, b):
        return pltpu.async_copy(
            rows_v.at[b], acc_sh.at[s2d.at[k]], ssems[b], add=True)

    def swait(k, b):
        pltpu.make_async_copy(
            rows_v.at[b], acc_sh.at[s2d.at[k]], ssems[b]).wait()

    for blk in range(_NBLK):
        base = pl.multiple_of(s * _PER_TILE + blk * _BIDX, 8)
        pltpu.sync_copy(gidx_hbm.at[pl.ds(base, _BIDX)], g1d)
        pltpu.sync_copy(sidx_hbm.at[pl.ds(base, _BIDX)], s1d)
        _adjust_idx(g1d, goff)
        _spread_idx(s1d, s2d)

        for b in range(_DR):
            gather(b, b)

        def body(i, carry):
            k0 = i * _DR
            for b in range(_DR):
                gwait(k0 + b, b)
                scat(k0 + b, b)
            for b in range(_DR):
                swait(k0 + b, b)
                gather(k0 + _DR + b, b)
            return carry
        lax.fori_loop(0, _BITERS, body, 0)

        kl = _BITERS * _DR
        for b in range(_DR):
            gwait(kl + b, b)
            scat(kl + b, b)
        for b in range(_DR):
            swait(kl + b, b)


def _make_hconv(n_slices):
    n_passes = n_slices // 2

    sem_types = [pltpu.SemaphoreType.DMA] * (2 * _DR + 1)

    @functools.partial(
        pl.kernel,
        out_type=jax.ShapeDtypeStruct((n_slices * _N_HEDGES, _F), jnp.float32),
        mesh=_mesh,
        scratch_types=[
            pltpu.VMEM((_BIDX,), jnp.int32),
            pltpu.VMEM((_BIDX,), jnp.int32),
            pltpu.VMEM((_BCHUNKS, _K), jnp.int32),
            pltpu.VMEM((_DR, _K, _F), jnp.float32),
            pltpu.VMEM((_ZR, _F), jnp.float32),
            pltpu.VMEM_SHARED((_N_HEDGES, _F), jnp.float32),
        ] + sem_types,
        compiler_params=pltpu.CompilerParams(use_tc_tiling_on_sc=False),
    )
    def hconv_s1(xs_ref, incn_ref, inch_ref, agge_hbm,
                 gidx_1d, sidx_1d, sidx_2d, rows_v, zero_v, agge_sh, *sems):
        c = lax.axis_index("c")
        s = lax.axis_index("s")
        gsems, ssems, msem = sems[:_DR], sems[_DR:2 * _DR], sems[2 * _DR]

        def zinit(i, carry):
            for f4 in range(_F // 16):
                zero_v[i, pl.ds(f4 * 16, 16)] = jnp.zeros((16,), jnp.float32)
            return carry
        lax.fori_loop(0, _ZR, zinit, 0)

        for p in range(n_passes):
            slice_id = 2 * p + c
            _zero_acc(zero_v, agge_sh, s, _EROWS, msem)
            plsc.subcore_barrier()

            _stage(xs_ref, agge_sh, incn_ref, inch_ref,
                   slice_id * _N_NODES, s,
                   gidx_1d, sidx_1d, sidx_2d, rows_v, gsems, ssems)
            plsc.subcore_barrier()

            pltpu.sync_copy(
                agge_sh.at[pl.ds(s * _EWROWS, _EWROWS)],
                agge_hbm.at[pl.ds(slice_id * _N_HEDGES + s * _EWROWS,
                                  _EWROWS)])

            @pl.when(s == 0)
            def _():
                pltpu.sync_copy(
                    agge_sh.at[pl.ds(16 * _EWROWS, _EWTAIL)],
                    agge_hbm.at[pl.ds(slice_id * _N_HEDGES + 16 * _EWROWS,
                                      _EWTAIL)])
            plsc.subcore_barrier()

    @functools.partial(
        pl.kernel,
        out_type=jax.ShapeDtypeStruct((n_slices * _N_NODES, _F), jnp.float32),
        mesh=_mesh,
        scratch_types=[
            pltpu.VMEM((_BIDX,), jnp.int32),
            pltpu.VMEM((_BIDX,), jnp.int32),
            pltpu.VMEM((_BCHUNKS, _K), jnp.int32),
            pltpu.VMEM((_DR, _K, _F), jnp.float32),
            pltpu.VMEM((_ZR, _F), jnp.float32),
            pltpu.VMEM_SHARED((_N_NODES, _F), jnp.float32),
        ] + sem_types,
        compiler_params=pltpu.CompilerParams(use_tc_tiling_on_sc=False),
    )
    def hconv_s2(agge_hbm, incn_ref, inch_ref, out_ref,
                 gidx_1d, sidx_1d, sidx_2d, rows_v, zero_v, aggn_sh, *sems):
        c = lax.axis_index("c")
        s = lax.axis_index("s")
        gsems, ssems, msem = sems[:_DR], sems[_DR:2 * _DR], sems[2 * _DR]

        def zinit(i, carry):
            for f4 in range(_F // 16):
                zero_v[i, pl.ds(f4 * 16, 16)] = jnp.zeros((16,), jnp.float32)
            return carry
        lax.fori_loop(0, _ZR, zinit, 0)

        for p in range(n_passes):
            slice_id = 2 * p + c
            _zero_acc(zero_v, aggn_sh, s, _NROWS, msem)
            plsc.subcore_barrier()

            _stage(agge_hbm, aggn_sh, inch_ref, incn_ref,
                   slice_id * _N_HEDGES, s,
                   gidx_1d, sidx_1d, sidx_2d, rows_v, gsems, ssems)
            plsc.subcore_barrier()

            pltpu.sync_copy(
                aggn_sh.at[pl.ds(s * _WROWS, _WROWS)],
                out_ref.at[pl.ds(slice_id * _N_NODES + s * _WROWS, _WROWS)])

            @pl.when(s == 0)
            def _():
                pltpu.sync_copy(
                    aggn_sh.at[pl.ds(16 * _WROWS, _WTAIL)],
                    out_ref.at[pl.ds(slice_id * _N_NODES + 16 * _WROWS,
                                     _WTAIL)])
            plsc.subcore_barrier()

    def hconv(xs, incn3, inch3):
        agge = hconv_s1(xs, incn3, inch3)
        return hconv_s2(agge, incn3, inch3)

    hconv.s1 = hconv_s1
    hconv.s2 = hconv_s2
    return hconv


_hconv2 = _make_hconv(2)
_hconv4 = _make_hconv(4)


_CG = _N_MEM // 32
_QG = 16


@functools.partial(
    pl.kernel,
    out_type=[
        jax.ShapeDtypeStruct((_N_MEM, _D), jnp.float32),
        jax.ShapeDtypeStruct((_Q, _D), jnp.float32),
    ],
    mesh=_mesh,
    scratch_types=[
        pltpu.VMEM((_CG,), jnp.int32),
        pltpu.VMEM((_CG, _D), jnp.float32),
        pltpu.VMEM((_QG,), jnp.int32),
        pltpu.VMEM((_QG, _D), jnp.float32),
        pltpu.SemaphoreType.DMA,
    ],
)
def _gather_rows(table_ref, cidx_ref, qidx_ref, outc_ref, outq_ref,
                 cidx_v, crow_v, qidx_v, qrow_v, sem):
    c = lax.axis_index("c")
    s = lax.axis_index("s")
    wid = s * 2 + c
    pltpu.sync_copy(cidx_ref.at[pl.ds(wid * _CG, _CG)], cidx_v)
    pltpu.async_copy(table_ref.at[cidx_v], crow_v, sem).wait()
    pltpu.sync_copy(crow_v, outc_ref.at[pl.ds(wid * _CG, _CG)])

    @pl.when(wid < _Q // _QG)
    def _():
        pltpu.sync_copy(qidx_ref.at[pl.ds(wid * _QG, _QG)], qidx_v)
        pltpu.async_copy(table_ref.at[qidx_v], qrow_v, sem).wait()
        pltpu.sync_copy(qrow_v, outq_ref.at[pl.ds(wid * _QG, _QG)])


_R = 1000


def _tc1_body(x_ref, w_ref, b_ref, out_ref):
    xv = x_ref[...]
    w = w_ref[...]
    acc = (jnp.dot(xv[0], w[0:_F, :], preferred_element_type=jnp.float32)
           + jnp.dot(xv[1], w[_F:2 * _F, :], preferred_element_type=jnp.float32)
           + b_ref[...])
    h = jnp.maximum(acc, 0.0)
    for sl in range(4):
        out_ref[sl] = h[:, sl * _F:(sl + 1) * _F]


def _tc1(aggn1, W1, b1):
    x3 = aggn1.reshape(2, _N_NODES, _F)
    return pl.pallas_call(
        _tc1_body,
        grid=(_N_NODES // _R,),
        in_specs=[
            pl.BlockSpec((2, _R, _F), lambda i: (0, i, 0)),
            pl.BlockSpec((_D, _HID), lambda i: (0, 0)),
            pl.BlockSpec((1, _HID), lambda i: (0, 0)),
        ],
        out_specs=pl.BlockSpec((4, _R, _F), lambda i: (0, i, 0)),
        out_shape=jax.ShapeDtypeStruct((4, _N_NODES, _F), jnp.float32),
    )(x3, W1, b1.reshape(1, _HID))


def _tc2_body(x_ref, w_ref, b_ref, out_ref):
    xv = x_ref[...]
    w = w_ref[...]
    acc = b_ref[...]
    for sl in range(4):
        acc = acc + jnp.dot(xv[sl], w[sl * _F:(sl + 1) * _F, :],
                            preferred_element_type=jnp.float32)
    out_ref[...] = jnp.maximum(acc, 0.0)


def _tc2(aggn2, W2, b2):
    x3 = aggn2.reshape(4, _N_NODES, _F)
    return pl.pallas_call(
        _tc2_body,
        grid=(_N_NODES // _R,),
        in_specs=[
            pl.BlockSpec((4, _R, _F), lambda i: (0, i, 0)),
            pl.BlockSpec((_HID, _D), lambda i: (0, 0)),
            pl.BlockSpec((1, _D), lambda i: (0, 0)),
        ],
        out_specs=pl.BlockSpec((_R, _D), lambda i: (i, 0)),
        out_shape=jax.ShapeDtypeStruct((_N_NODES, _D), jnp.float32),
    )(x3, W2, b2.reshape(1, _D))


def _score_body(x0q_ref, rfq_ref, x0m_ref, rfm_ref, ids_ref,
                phiW_ref, phib_ref, rhoW_ref, rhob_ref,
                qpW_ref, qpb_ref, g1W_ref, g1b_ref, g2W_ref, g2b_ref,
                m1W_ref, m1b_ref, m2W_ref, m2b_ref, out_ref):
    qpW = qpW_ref[...]
    qpb = qpb_ref[...]
    base_q = jnp.dot(x0q_ref[...], qpW, preferred_element_type=jnp.float32) + qpb
    topo_q = jnp.dot(rfq_ref[...], qpW, preferred_element_type=jnp.float32) + qpb

    gh = jnp.maximum(
        jnp.dot(topo_q, g1W_ref[...], preferred_element_type=jnp.float32)
        + g1b_ref[...], 0.0)
    glog = jnp.dot(gh, g2W_ref[...], preferred_element_type=jnp.float32) + g2b_ref[...]
    gate = 1.0 / (1.0 + jnp.exp(-glog))

    phiW = phiW_ref[...]
    phib = phib_ref[...]
    hb = jnp.maximum(
        jnp.dot(x0m_ref[...], phiW, preferred_element_type=jnp.float32) + phib, 0.0)
    hr = jnp.maximum(
        jnp.dot(rfm_ref[...], phiW, preferred_element_type=jnp.float32) + phib, 0.0)

    mi = lax.broadcasted_iota(jnp.int32, (_M, _N_MEM), 0)
    sel = (mi == ids_ref[...]).astype(jnp.float32)
    pooled_b = jnp.dot(sel, hb, preferred_element_type=jnp.float32)
    pooled_r = jnp.dot(sel, hr, preferred_element_type=jnp.float32)

    rhoW = rhoW_ref[...]
    rhob = rhob_ref[...]
    base_c = jnp.dot(pooled_b, rhoW, preferred_element_type=jnp.float32) + rhob
    topo_c = jnp.dot(pooled_r, rhoW, preferred_element_type=jnp.float32) + rhob

    m1W = m1W_ref[...]
    q_final = (1.0 - gate) * base_q + gate * topo_q
    qh = jnp.dot(q_final, m1W[:_HID, :], preferred_element_type=jnp.float32) + m1b_ref[...]
    BCc = jnp.dot(base_c, m1W[_HID:, :], preferred_element_type=jnp.float32)
    TCc = jnp.dot(topo_c, m1W[_HID:, :], preferred_element_type=jnp.float32)

    w2 = m2W_ref[...][:, 0]
    b2s = m2b_ref[0, 0]
    QB = 8
    for qb in range(_Q // QB):
        lo, hi = qb * QB, (qb + 1) * QB
        g = gate[lo:hi]
        hq = (qh[lo:hi][:, None, :]
              + (1.0 - g)[:, :, None] * BCc[None, :, :]
              + g[:, :, None] * TCc[None, :, :])
        sc = jnp.sum(jnp.maximum(hq, 0.0) * w2[None, None, :], axis=-1) + b2s
        out_ref[lo:hi] = sc


def _score(x0q, rfq, x0m, rfm, ids, phi_W, phi_b, rho_W, rho_b,
           qp_W, qp_b, g1_W, g1_b, g2_W, g2_b, mlp1_W, mlp1_b, mlp2_W, mlp2_b):
    return pl.pallas_call(
        _score_body,
        out_shape=jax.ShapeDtypeStruct((_Q, _M), jnp.float32),
    )(x0q, rfq, x0m, rfm, ids.reshape(1, _N_MEM),
      phi_W, phi_b.reshape(1, _HID), rho_W, rho_b.reshape(1, _HID),
      qp_W, qp_b.reshape(1, _HID), g1_W, g1_b.reshape(1, _HID),
      g2_W, g2_b.reshape(1, 1),
      mlp1_W, mlp1_b.reshape(1, _HID), mlp2_W, mlp2_b.reshape(1, 1))


def kernel(x_0, inc_node, inc_hedge, query_node_indices, cell_member_node,
           cell_member_cell, W1, b1, W2, b2, phi_W, phi_b, rho_W, rho_b,
           qp_W, qp_b, g1_W, g1_b, g2_W, g2_b, mlp1_W, mlp1_b, mlp2_W, mlp2_b):
    incn = inc_node.astype(jnp.int32)
    inch = inc_hedge.astype(jnp.int32)
    cidx = cell_member_node.astype(jnp.int32)
    qidx = query_node_indices.astype(jnp.int32)
    ids = cell_member_cell.astype(jnp.int32)

    incn = incn.reshape(_N_TILES, _K, _CHUNKS).transpose(0, 2, 1).reshape(-1)
    inch = inch.reshape(_N_TILES, _K, _CHUNKS).transpose(0, 2, 1).reshape(-1)

    xs1 = x_0.reshape(_N_NODES, 2, _F).transpose(1, 0, 2).reshape(2 * _N_NODES, _F)
    aggn1 = _hconv2(xs1, incn, inch)
    h1s = _tc1(aggn1, W1, b1).reshape(4 * _N_NODES, _F)
    aggn2 = _hconv4(h1s, incn, inch)
    refined = _tc2(aggn2, W2, b2)

    x0m, x0q = _gather_rows(x_0, cidx, qidx)
    rfm, rfq = _gather_rows(refined, cidx, qidx)

    return _score(x0q, rfq, x0m, rfm, ids, phi_W, phi_b, rho_W, rho_b,
                  qp_W, qp_b, g1_W, g1_b, g2_W, g2_b,
                  mlp1_W, mlp1_b, mlp2_W, mlp2_b)
